# Initial kernel scaffold; baseline (speedup 1.0000x reference)
import jax
import jax.numpy as jnp
from jax.experimental import pallas as pl


def kernel(x, edge_index, edge_attr, u, batch, W1a, b1a, W2a, b2a, W1n, b1n, W2n, b2n, gamma, beta):
    raise NotImplementedError("write your pallas kernel here")


# speedup vs baseline: 5.7479x; 5.7479x over previous
import jax
import jax.numpy as jnp
from jax import lax
from jax.experimental import pallas as pl
from jax.experimental.pallas import tpu as pltpu
from jax.experimental.pallas import tpu_sc as plsc

N = 10000
E = 320000
D = 128
DE = 4
DG = 16
G = 64
H1A = 4 * DE
H1N = 4 * D

NC = 2
NS = 16
NP = 10240
EPC = E // NC
KE = 2048

NB = 2048
NBLK = NP // NB
EB = E // NBLK


def _chunks():
    out = []
    off = 0
    while off < EPC:
        sz = min(KE, EPC - off)
        out.append((off, sz))
        off += sz
    return out


def _pre_body(x_ref, eat_ref, b_ref, w1ax_ref, b1a_ref, w1ae_ref,
              w1nx_ref, u_ref, w1nu_ref, b1n_ref,
              pt_ref, epret_ref, z0_ref):
    x = x_ref[...]
    pt_ref[...] = lax.dot_general(
        w1ax_ref[...], x, (((0,), (1,)), ((), ()))) + b1a_ref[...]
    epret_ref[...] = lax.dot_general(
        w1ae_ref[...], eat_ref[...], (((0,), (0,)), ((), ())))
    oh = (b_ref[...] == lax.broadcasted_iota(jnp.int32, (1, G), 1)
          ).astype(jnp.float32)
    uw = u_ref[...] @ w1nu_ref[...]
    z0_ref[...] = x @ w1nx_ref[...] + oh @ uw + b1n_ref[...]


def _tc_pre(x, eat, batch2, w1ax, b1a, w1ae, w1nx, u, w1nu, b1n):
    row_blk = lambda i: (i, 0)
    col_blk = lambda i: (0, i)
    full = lambda i: (0, 0)
    return pl.pallas_call(
        _pre_body,
        grid=(NBLK,),
        in_specs=[
            pl.BlockSpec((NB, D), row_blk),
            pl.BlockSpec((DE, EB), col_blk),
            pl.BlockSpec((NB, 1), row_blk),
            pl.BlockSpec((D, H1A), full),
            pl.BlockSpec((H1A, 1), full),
            pl.BlockSpec((DE, H1A), full),
            pl.BlockSpec((D, H1N), full),
            pl.BlockSpec((G, DG), full),
            pl.BlockSpec((DG, H1N), full),
            pl.BlockSpec((1, H1N), full),
        ],
        out_specs=[
            pl.BlockSpec((H1A, NB), col_blk),
            pl.BlockSpec((H1A, EB), col_blk),
            pl.BlockSpec((NB, H1N), row_blk),
        ],
        out_shape=[
            jax.ShapeDtypeStruct((H1A, NP), jnp.float32),
            jax.ShapeDtypeStruct((H1A, E), jnp.float32),
            jax.ShapeDtypeStruct((NP, H1N), jnp.float32),
        ],
    )(x, eat, batch2, w1ax, b1a, w1ae, w1nx, u, w1nu, b1n)


def _sc_body(pt_hbm, epret_hbm, row_hbm, col_hbm, out_hbm,
             pcol, acc, cnt, idx_v, col_v, ep_v):
    c = lax.axis_index("c")
    s = lax.axis_index("s")

    zero16 = jnp.zeros((16,), jnp.float32)
    ones16 = jnp.full((16,), 1.0, jnp.float32)

    def zero_body(j, carry):
        acc[pl.ds(j * 16, 16)] = zero16
        cnt[pl.ds(j * 16, 16)] = zero16
        return carry
    lax.fori_loop(0, NP // 16, zero_body, 0)

    pltpu.sync_copy(pt_hbm.at[pl.ds(s * NP, NP)], pcol)

    ebase = c * EPC
    for gi, (off, sz) in enumerate(_chunks()):
        base = ebase + off
        pltpu.sync_copy(row_hbm.at[pl.ds(base, sz)], idx_v.at[pl.ds(0, sz)])
        pltpu.sync_copy(col_hbm.at[pl.ds(base, sz)], col_v.at[pl.ds(0, sz)])
        pltpu.sync_copy(epret_hbm.at[pl.ds(s * E + base, sz)],
                        ep_v.at[pl.ds(0, sz)])

        def ew(i, carry):
            o = i * 16
            i16 = idx_v[pl.ds(o, 16)]
            c16 = col_v[pl.ds(o, 16)]
            e16 = ep_v[pl.ds(o, 16)]
            g16 = plsc.load_gather(pcol, [i16])
            plsc.addupdate_scatter(acc, [c16], jnp.maximum(g16 + e16, 0.0))
            return carry
        lax.fori_loop(0, sz // 16, ew, 0)

        @pl.when(s == gi % NS)
        def _count():
            def cw(i, carry):
                c16 = col_v[pl.ds(i * 16, 16)]
                plsc.addupdate_scatter(cnt, [c16], ones16)
                return carry
            lax.fori_loop(0, sz // 16, cw, 0)

    pltpu.sync_copy(acc, out_hbm.at[pl.ds((c * 32 + s) * NP, NP)])
    pltpu.sync_copy(cnt, out_hbm.at[pl.ds((c * 32 + 16 + s) * NP, NP)])


def _sc_edge(pt_flat, epret_flat, row, col):
    run = pl.kernel(
        _sc_body,
        out_type=jax.ShapeDtypeStruct((NC * 32 * NP,), jnp.float32),
        mesh=plsc.VectorSubcoreMesh(core_axis_name="c", subcore_axis_name="s",
                                    num_cores=NC, num_subcores=NS),
        scratch_types=[
            pltpu.VMEM((NP,), jnp.float32),
            pltpu.VMEM((NP,), jnp.float32),
            pltpu.VMEM((NP,), jnp.float32),
            pltpu.VMEM((KE,), jnp.int32),
            pltpu.VMEM((KE,), jnp.int32),
            pltpu.VMEM((KE,), jnp.float32),
        ],
        compiler_params=pltpu.CompilerParams(needs_layout_passes=False),
    )
    return run(pt_flat, epret_flat, row, col)


def _post_body(a2_ref, z0_ref, x_ref, w2a_ref, b2a_ref, w1na_ref,
               w2n_ref, b2n_ref, gamma_ref, beta_ref, out_ref):
    a = a2_ref[0] + a2_ref[1]
    st = a[:H1A, :]
    cntt = jnp.sum(a[H1A:2 * H1A, :], axis=0, keepdims=True)
    inv = 1.0 / jnp.maximum(cntt, 1.0)
    w2ap = w2a_ref[...] @ w1na_ref[...]
    b2ap = b2a_ref[...] @ w1na_ref[...]
    m = lax.dot_general(st * inv, w2ap, (((0,), (0,)), ((), ())))
    bterm = lax.dot_general(cntt * inv, b2ap, (((0,), (0,)), ((), ())))
    z1 = z0_ref[...] + m + bterm
    h = jnp.maximum(z1, 0.0)
    y = h @ w2n_ref[...] + b2n_ref[...] + x_ref[...]
    mu = jnp.mean(y, axis=1, keepdims=True)
    yc = y - mu
    var = jnp.mean(yc * yc, axis=1, keepdims=True)
    out_ref[...] = yc * lax.rsqrt(var + 1e-5) * gamma_ref[...] + beta_ref[...]


def _tc_post(a2, z0, x, w2a, b2a, w1na, w2n, b2n, gamma, beta):
    row_blk = lambda i: (i, 0)
    full = lambda i: (0, 0)
    return pl.pallas_call(
        _post_body,
        grid=(NBLK,),
        in_specs=[
            pl.BlockSpec((NC, 32, NB), lambda i: (0, 0, i)),
            pl.BlockSpec((NB, H1N), row_blk),
            pl.BlockSpec((NB, D), row_blk),
            pl.BlockSpec((H1A, DE), full),
            pl.BlockSpec((1, DE), full),
            pl.BlockSpec((DE, H1N), full),
            pl.BlockSpec((H1N, D), full),
            pl.BlockSpec((1, D), full),
            pl.BlockSpec((1, D), full),
            pl.BlockSpec((1, D), full),
        ],
        out_specs=pl.BlockSpec((NB, D), row_blk),
        out_shape=jax.ShapeDtypeStruct((NP, D), jnp.float32),
    )(a2, z0, x, w2a, b2a, w1na, w2n, b2n, gamma, beta)


def kernel(x, edge_index, edge_attr, u, batch,
           W1a, b1a, W2a, b2a, W1n, b1n, W2n, b2n, gamma, beta):
    row = edge_index[0].astype(jnp.int32)
    col = edge_index[1].astype(jnp.int32)

    w1ax = W1a[:D]
    w1ae = W1a[D:]
    w1nx = W1n[:D]
    w1na = W1n[D:D + DE]
    w1nu = W1n[D + DE:]

    eat = edge_attr.T
    xp = jnp.pad(x, ((0, NP - N), (0, 0)))
    batch2 = jnp.pad(batch.astype(jnp.int32), (0, NP - N)).reshape(NP, 1)

    pt, epret, z0 = _tc_pre(
        xp, eat, batch2, w1ax, b1a.reshape(H1A, 1), w1ae,
        w1nx, u, w1nu, b1n.reshape(1, H1N))

    out_flat = _sc_edge(pt.reshape(H1A * NP), epret.reshape(H1A * E),
                        row, col)
    a2 = out_flat.reshape(NC, 32, NP)

    outp = _tc_post(a2, z0, xp, W2a, b2a.reshape(1, DE), w1na,
                    W2n, b2n.reshape(1, D), gamma.reshape(1, D),
                    beta.reshape(1, D))
    return outp[:N]


# speedup vs baseline: 13.8973x; 2.4178x over previous
import jax
import jax.numpy as jnp
from jax import lax
from jax.experimental import pallas as pl
from jax.experimental.pallas import tpu as pltpu
from jax.experimental.pallas import tpu_sc as plsc

N = 10000
E = 320000
D = 128
DE = 4
DG = 16
G = 64
H1A = 4 * DE
H1N = 4 * D

NC = 2
NS = 16
NP = 10240
EPC = E // NC
KE = 3200
NCH = EPC // KE

NB = 2048
NBLK = NP // NB
EB = E // NBLK


def _pre_body(x_ref, eat_ref, b_ref, w1ax_ref, b1a_ref, w1ae_ref,
              w1nx_ref, u_ref, w1nu_ref, b1n_ref,
              pt_ref, epret_ref, z0_ref):
    x = x_ref[...]
    pt_ref[...] = lax.dot_general(
        w1ax_ref[...], x, (((0,), (1,)), ((), ()))) + b1a_ref[...]
    epret_ref[...] = lax.dot_general(
        w1ae_ref[...], eat_ref[...], (((0,), (0,)), ((), ())))
    oh = (b_ref[...] == lax.broadcasted_iota(jnp.int32, (1, G), 1)
          ).astype(jnp.float32)
    uw = u_ref[...] @ w1nu_ref[...]
    z0_ref[...] = x @ w1nx_ref[...] + oh @ uw + b1n_ref[...]


def _tc_pre(x, eat, batch2, w1ax, b1a, w1ae, w1nx, u, w1nu, b1n):
    row_blk = lambda i: (i, 0)
    col_blk = lambda i: (0, i)
    full = lambda i: (0, 0)
    return pl.pallas_call(
        _pre_body,
        grid=(NBLK,),
        in_specs=[
            pl.BlockSpec((NB, D), row_blk),
            pl.BlockSpec((DE, EB), col_blk),
            pl.BlockSpec((NB, 1), row_blk),
            pl.BlockSpec((D, H1A), full),
            pl.BlockSpec((H1A, 1), full),
            pl.BlockSpec((DE, H1A), full),
            pl.BlockSpec((D, H1N), full),
            pl.BlockSpec((G, DG), full),
            pl.BlockSpec((DG, H1N), full),
            pl.BlockSpec((1, H1N), full),
        ],
        out_specs=[
            pl.BlockSpec((H1A, NB), col_blk),
            pl.BlockSpec((H1A, EB), col_blk),
            pl.BlockSpec((NB, H1N), row_blk),
        ],
        out_shape=[
            jax.ShapeDtypeStruct((H1A, NP), jnp.float32),
            jax.ShapeDtypeStruct((H1A, E), jnp.float32),
            jax.ShapeDtypeStruct((NP, H1N), jnp.float32),
        ],
    )(x, eat, batch2, w1ax, b1a, w1ae, w1nx, u, w1nu, b1n)


def _sc_body(pt_hbm, epret_hbm, row_hbm, col_hbm, out_hbm,
             pcol, acc, cnt,
             idx_a, col_a, ep_a, idx_b, col_b, ep_b,
             psem, sem_a, sem_b):
    c = lax.axis_index("c")
    s = lax.axis_index("s")

    zero16 = jnp.zeros((16,), jnp.float32)
    ones16 = jnp.full((16,), 1.0, jnp.float32)

    ph = pltpu.async_copy(pt_hbm.at[pl.ds(s * NP, NP)], pcol, psem)

    ebase = c * EPC
    bufs = ((idx_a, col_a, ep_a, sem_a), (idx_b, col_b, ep_b, sem_b))

    def start(g, buf):
        base = ebase + jnp.minimum(g, NCH - 1) * KE
        iv, cv, ev, sm = buf
        return (
            pltpu.async_copy(row_hbm.at[pl.ds(base, KE)], iv, sm),
            pltpu.async_copy(col_hbm.at[pl.ds(base, KE)], cv, sm),
            pltpu.async_copy(epret_hbm.at[pl.ds(s * E + base, KE)], ev, sm),
        )

    def wait(buf):
        iv, cv, ev, sm = buf
        pltpu.make_async_copy(row_hbm.at[pl.ds(0, KE)], iv, sm).wait()
        pltpu.make_async_copy(col_hbm.at[pl.ds(0, KE)], cv, sm).wait()
        pltpu.make_async_copy(epret_hbm.at[pl.ds(0, KE)], ev, sm).wait()

    start(0, bufs[0])
    start(1, bufs[1])

    @plsc.parallel_loop(0, NP // 16, 1, unroll=4)
    def _zero(j):
        acc[pl.ds(j * 16, 16)] = zero16
        cnt[pl.ds(j * 16, 16)] = zero16

    ph.wait()

    def compute(g, buf):
        iv, cv, ev, _ = buf
        wait(buf)

        @plsc.parallel_loop(0, KE // 16, 1, unroll=8)
        def _ew(i):
            o = i * 16
            i16 = iv[pl.ds(o, 16)]
            c16 = cv[pl.ds(o, 16)]
            e16 = ev[pl.ds(o, 16)]
            g16 = plsc.load_gather(pcol, [i16])
            plsc.addupdate_scatter(acc, [c16], jnp.maximum(g16 + e16, 0.0))

        @pl.when(s == g % NS)
        def _count():
            @plsc.parallel_loop(0, KE // 16, 1, unroll=4)
            def _cw(i):
                c16 = cv[pl.ds(i * 16, 16)]
                plsc.addupdate_scatter(cnt, [c16], ones16)

    @pl.loop(0, NCH // 2)
    def _pair(j):
        g = j * 2
        compute(g, bufs[0])
        start(g + 2, bufs[0])
        compute(g + 1, bufs[1])
        start(g + 3, bufs[1])

    wait(bufs[0])
    wait(bufs[1])

    pltpu.sync_copy(acc, out_hbm.at[pl.ds((c * 32 + s) * NP, NP)])
    pltpu.sync_copy(cnt, out_hbm.at[pl.ds((c * 32 + 16 + s) * NP, NP)])


def _sc_edge(pt_flat, epret_flat, row, col):
    run = pl.kernel(
        _sc_body,
        out_type=jax.ShapeDtypeStruct((NC * 32 * NP,), jnp.float32),
        mesh=plsc.VectorSubcoreMesh(core_axis_name="c", subcore_axis_name="s",
                                    num_cores=NC, num_subcores=NS),
        scratch_types=[
            pltpu.VMEM((NP,), jnp.float32),
            pltpu.VMEM((NP,), jnp.float32),
            pltpu.VMEM((NP,), jnp.float32),
            pltpu.VMEM((KE,), jnp.int32),
            pltpu.VMEM((KE,), jnp.int32),
            pltpu.VMEM((KE,), jnp.float32),
            pltpu.VMEM((KE,), jnp.int32),
            pltpu.VMEM((KE,), jnp.int32),
            pltpu.VMEM((KE,), jnp.float32),
            pltpu.SemaphoreType.DMA,
            pltpu.SemaphoreType.DMA,
            pltpu.SemaphoreType.DMA,
        ],
        compiler_params=pltpu.CompilerParams(needs_layout_passes=False),
    )
    return run(pt_flat, epret_flat, row, col)


def _post_body(a2_ref, z0_ref, x_ref, w2a_ref, b2a_ref, w1na_ref,
               w2n_ref, b2n_ref, gamma_ref, beta_ref, out_ref):
    a = a2_ref[0] + a2_ref[1]
    st = a[:H1A, :]
    cntt = jnp.sum(a[H1A:2 * H1A, :], axis=0, keepdims=True)
    inv = 1.0 / jnp.maximum(cntt, 1.0)
    w2ap = w2a_ref[...] @ w1na_ref[...]
    b2ap = b2a_ref[...] @ w1na_ref[...]
    m = lax.dot_general(st * inv, w2ap, (((0,), (0,)), ((), ())))
    bterm = lax.dot_general(cntt * inv, b2ap, (((0,), (0,)), ((), ())))
    z1 = z0_ref[...] + m + bterm
    h = jnp.maximum(z1, 0.0)
    y = h @ w2n_ref[...] + b2n_ref[...] + x_ref[...]
    mu = jnp.mean(y, axis=1, keepdims=True)
    yc = y - mu
    var = jnp.mean(yc * yc, axis=1, keepdims=True)
    out_ref[...] = yc * lax.rsqrt(var + 1e-5) * gamma_ref[...] + beta_ref[...]


def _tc_post(a2, z0, x, w2a, b2a, w1na, w2n, b2n, gamma, beta):
    row_blk = lambda i: (i, 0)
    full = lambda i: (0, 0)
    return pl.pallas_call(
        _post_body,
        grid=(NBLK,),
        in_specs=[
            pl.BlockSpec((NC, 32, NB), lambda i: (0, 0, i)),
            pl.BlockSpec((NB, H1N), row_blk),
            pl.BlockSpec((NB, D), row_blk),
            pl.BlockSpec((H1A, DE), full),
            pl.BlockSpec((1, DE), full),
            pl.BlockSpec((DE, H1N), full),
            pl.BlockSpec((H1N, D), full),
            pl.BlockSpec((1, D), full),
            pl.BlockSpec((1, D), full),
            pl.BlockSpec((1, D), full),
        ],
        out_specs=pl.BlockSpec((NB, D), row_blk),
        out_shape=jax.ShapeDtypeStruct((NP, D), jnp.float32),
    )(a2, z0, x, w2a, b2a, w1na, w2n, b2n, gamma, beta)


def kernel(x, edge_index, edge_attr, u, batch,
           W1a, b1a, W2a, b2a, W1n, b1n, W2n, b2n, gamma, beta):
    row = edge_index[0].astype(jnp.int32)
    col = edge_index[1].astype(jnp.int32)

    w1ax = W1a[:D]
    w1ae = W1a[D:]
    w1nx = W1n[:D]
    w1na = W1n[D:D + DE]
    w1nu = W1n[D + DE:]

    eat = edge_attr.T
    xp = jnp.pad(x, ((0, NP - N), (0, 0)))
    batch2 = jnp.pad(batch.astype(jnp.int32), (0, NP - N)).reshape(NP, 1)

    pt, epret, z0 = _tc_pre(
        xp, eat, batch2, w1ax, b1a.reshape(H1A, 1), w1ae,
        w1nx, u, w1nu, b1n.reshape(1, H1N))

    out_flat = _sc_edge(pt.reshape(H1A * NP), epret.reshape(H1A * E),
                        row, col)
    a2 = out_flat.reshape(NC, 32, NP)

    outp = _tc_post(a2, z0, xp, W2a, b2a.reshape(1, DE), w1na,
                    W2n, b2n.reshape(1, D), gamma.reshape(1, D),
                    beta.reshape(1, D))
    return outp[:N]


# speedup vs baseline: 18.0493x; 1.2988x over previous
import jax
import jax.numpy as jnp
from jax import lax
from jax.experimental import pallas as pl
from jax.experimental.pallas import tpu as pltpu
from jax.experimental.pallas import tpu_sc as plsc

N = 10000
E = 320000
D = 128
DE = 4
DG = 16
G = 64
H1A = 4 * DE
H1N = 4 * D

NC = 2
NS = 16
NP = 10240
EPC = E // NC
KE = 6400
NCH = EPC // KE

NB = 2048
NBLK = NP // NB
EB = E // NBLK


def _pre_body(x_ref, eat_ref, w1ax_ref, b1a_ref, w1ae_ref,
              pt_ref, epret_ref):
    pt_ref[...] = lax.dot_general(
        w1ax_ref[...], x_ref[...], (((0,), (1,)), ((), ()))) + b1a_ref[...]
    epret_ref[...] = lax.dot_general(
        w1ae_ref[...], eat_ref[...],
        (((0,), (0,)), ((), ()))).astype(jnp.bfloat16)


def _tc_pre(x, eat, w1ax, b1a, w1ae):
    row_blk = lambda i: (i, 0)
    col_blk = lambda i: (0, i)
    full = lambda i: (0, 0)
    return pl.pallas_call(
        _pre_body,
        grid=(NBLK,),
        in_specs=[
            pl.BlockSpec((NB, D), row_blk),
            pl.BlockSpec((DE, EB), col_blk),
            pl.BlockSpec((D, H1A), full),
            pl.BlockSpec((H1A, 1), full),
            pl.BlockSpec((DE, H1A), full),
        ],
        out_specs=[
            pl.BlockSpec((H1A, NB), col_blk),
            pl.BlockSpec((H1A, EB), col_blk),
        ],
        out_shape=[
            jax.ShapeDtypeStruct((H1A, NP), jnp.float32),
            jax.ShapeDtypeStruct((H1A, E), jnp.bfloat16),
        ],
    )(x, eat, w1ax, b1a, w1ae)


def _sc_body(pt_hbm, epret_hbm, ec_hbm, out_hbm,
             pcol, acc, cnt,
             idx_a, col_a, ep_a, idx_b, col_b, ep_b,
             psem, sem_a, sem_b):
    c = lax.axis_index("c")
    s = lax.axis_index("s")

    zero16 = jnp.zeros((16,), jnp.float32)
    ones16 = jnp.full((16,), 1.0, jnp.float32)

    ph = pltpu.async_copy(pt_hbm.at[pl.ds(s * NP, NP)], pcol, psem)

    ebase = c * EPC
    bufs = ((idx_a, col_a, ep_a, sem_a), (idx_b, col_b, ep_b, sem_b))

    def start(g, buf):
        base = ebase + jnp.minimum(g, NCH - 1) * KE
        iv, cv, ev, sm = buf
        return (
            pltpu.async_copy(ec_hbm.at[pl.ds(base, KE)], iv, sm),
            pltpu.async_copy(ec_hbm.at[pl.ds(E + base, KE)], cv, sm),
            pltpu.async_copy(epret_hbm.at[pl.ds(s * E + base, KE)], ev, sm),
        )

    def wait(buf):
        iv, cv, ev, sm = buf
        pltpu.make_async_copy(ec_hbm.at[pl.ds(0, KE)], iv, sm).wait()
        pltpu.make_async_copy(ec_hbm.at[pl.ds(0, KE)], cv, sm).wait()
        pltpu.make_async_copy(epret_hbm.at[pl.ds(0, KE)], ev, sm).wait()

    start(0, bufs[0])
    start(1, bufs[1])

    @plsc.parallel_loop(0, NP // 16, 1, unroll=4)
    def _zero(j):
        acc[pl.ds(j * 16, 16)] = zero16
        cnt[pl.ds(j * 16, 16)] = zero16

    ph.wait()

    def compute(g, buf):
        iv, cv, ev, _ = buf
        wait(buf)

        @plsc.parallel_loop(0, KE // 256, 1, unroll=1)
        def _ew(t):
            for k in range(8):
                o = t * 256 + k * 16
                ia, ib = plsc.unpack(iv[pl.ds(o, 32)],
                                     format=plsc.PackFormat.INTERLEAVED,
                                     preferred_element_type=jnp.int32)
                ca, cb = plsc.unpack(cv[pl.ds(o, 32)],
                                     format=plsc.PackFormat.INTERLEAVED,
                                     preferred_element_type=jnp.int32)
                ea, eb = plsc.unpack(ev[pl.ds(o, 32)],
                                     format=plsc.PackFormat.INTERLEAVED,
                                     preferred_element_type=jnp.float32)
                ga = plsc.load_gather(pcol, [ia])
                plsc.addupdate_scatter(acc, [ca], jnp.maximum(ga + ea, 0.0))
                gb = plsc.load_gather(pcol, [ib])
                plsc.addupdate_scatter(acc, [cb], jnp.maximum(gb + eb, 0.0))

        @pl.when(s == g % NS)
        def _count():
            @plsc.parallel_loop(0, KE // 256, 1, unroll=1)
            def _cw(t):
                for k in range(8):
                    o = t * 256 + k * 16
                    ca, cb = plsc.unpack(cv[pl.ds(o, 32)],
                                         format=plsc.PackFormat.INTERLEAVED,
                                         preferred_element_type=jnp.int32)
                    plsc.addupdate_scatter(cnt, [ca], ones16)
                    plsc.addupdate_scatter(cnt, [cb], ones16)

    @pl.loop(0, NCH // 2)
    def _pair(j):
        g = j * 2
        compute(g, bufs[0])
        start(g + 2, bufs[0])
        compute(g + 1, bufs[1])
        start(g + 3, bufs[1])

    compute(NCH - 1, bufs[0])
    wait(bufs[1])

    pltpu.sync_copy(acc, out_hbm.at[pl.ds((c * 32 + s) * NP, NP)])
    pltpu.sync_copy(cnt, out_hbm.at[pl.ds((c * 32 + 16 + s) * NP, NP)])


def _sc_edge(pt_flat, epret_flat, ec):
    run = pl.kernel(
        _sc_body,
        out_type=jax.ShapeDtypeStruct((NC * 32 * NP,), jnp.float32),
        mesh=plsc.VectorSubcoreMesh(core_axis_name="c", subcore_axis_name="s",
                                    num_cores=NC, num_subcores=NS),
        scratch_types=[
            pltpu.VMEM((NP,), jnp.float32),
            pltpu.VMEM((NP,), jnp.float32),
            pltpu.VMEM((NP,), jnp.float32),
            pltpu.VMEM((KE,), jnp.int16),
            pltpu.VMEM((KE,), jnp.int16),
            pltpu.VMEM((KE,), jnp.bfloat16),
            pltpu.VMEM((KE,), jnp.int16),
            pltpu.VMEM((KE,), jnp.int16),
            pltpu.VMEM((KE,), jnp.bfloat16),
            pltpu.SemaphoreType.DMA,
            pltpu.SemaphoreType.DMA,
            pltpu.SemaphoreType.DMA,
        ],
        compiler_params=pltpu.CompilerParams(needs_layout_passes=False),
    )
    return run(pt_flat, epret_flat, ec)


def _post_body(a2_ref, x_ref, b_ref, w1nx_ref, u_ref, w1nu_ref, b1n_ref,
               w2a_ref, b2a_ref, w1na_ref,
               w2n_ref, b2n_ref, gamma_ref, beta_ref, out_ref):
    x = x_ref[...]
    a = a2_ref[0] + a2_ref[1]
    st = a[:H1A, :]
    cntt = jnp.sum(a[H1A:2 * H1A, :], axis=0, keepdims=True)
    inv = 1.0 / jnp.maximum(cntt, 1.0)
    w2ap = w2a_ref[...] @ w1na_ref[...]
    b2ap = b2a_ref[...] @ w1na_ref[...]
    m = lax.dot_general(st * inv, w2ap, (((0,), (0,)), ((), ())))
    bterm = lax.dot_general(cntt * inv, b2ap, (((0,), (0,)), ((), ())))
    oh = (b_ref[...] == lax.broadcasted_iota(jnp.int32, (1, G), 1)
          ).astype(jnp.float32)
    uw = u_ref[...] @ w1nu_ref[...]
    z1 = x @ w1nx_ref[...] + oh @ uw + b1n_ref[...] + m + bterm
    h = jnp.maximum(z1, 0.0)
    y = h @ w2n_ref[...] + b2n_ref[...] + x
    mu = jnp.mean(y, axis=1, keepdims=True)
    yc = y - mu
    var = jnp.mean(yc * yc, axis=1, keepdims=True)
    out_ref[...] = yc * lax.rsqrt(var + 1e-5) * gamma_ref[...] + beta_ref[...]


def _tc_post(a2, x, batch2, w1nx, u, w1nu, b1n, w2a, b2a, w1na,
             w2n, b2n, gamma, beta):
    row_blk = lambda i: (i, 0)
    full = lambda i: (0, 0)
    return pl.pallas_call(
        _post_body,
        grid=(NBLK,),
        in_specs=[
            pl.BlockSpec((NC, 32, NB), lambda i: (0, 0, i)),
            pl.BlockSpec((NB, D), row_blk),
            pl.BlockSpec((NB, 1), row_blk),
            pl.BlockSpec((D, H1N), full),
            pl.BlockSpec((G, DG), full),
            pl.BlockSpec((DG, H1N), full),
            pl.BlockSpec((1, H1N), full),
            pl.BlockSpec((H1A, DE), full),
            pl.BlockSpec((1, DE), full),
            pl.BlockSpec((DE, H1N), full),
            pl.BlockSpec((H1N, D), full),
            pl.BlockSpec((1, D), full),
            pl.BlockSpec((1, D), full),
            pl.BlockSpec((1, D), full),
        ],
        out_specs=pl.BlockSpec((NB, D), row_blk),
        out_shape=jax.ShapeDtypeStruct((NP, D), jnp.float32),
    )(a2, x, batch2, w1nx, u, w1nu, b1n, w2a, b2a, w1na,
      w2n, b2n, gamma, beta)


def kernel(x, edge_index, edge_attr, u, batch,
           W1a, b1a, W2a, b2a, W1n, b1n, W2n, b2n, gamma, beta):
    ec = edge_index.astype(jnp.int16).reshape(2 * E)

    w1ax = W1a[:D]
    w1ae = W1a[D:]
    w1nx = W1n[:D]
    w1na = W1n[D:D + DE]
    w1nu = W1n[D + DE:]

    xp = jnp.pad(x, ((0, NP - N), (0, 0)))
    batch2 = jnp.pad(batch.astype(jnp.int32), (0, NP - N)).reshape(NP, 1)

    eat = edge_attr.T
    pt, epret = _tc_pre(xp, eat, w1ax, b1a.reshape(H1A, 1), w1ae)

    out_flat = _sc_edge(pt.reshape(H1A * NP), epret.reshape(H1A * E), ec)
    a2 = out_flat.reshape(NC, 32, NP)

    outp = _tc_post(a2, xp, batch2, w1nx, u, w1nu, b1n.reshape(1, H1N),
                    W2a, b2a.reshape(1, DE), w1na,
                    W2n, b2n.reshape(1, D), gamma.reshape(1, D),
                    beta.reshape(1, D))
    return outp[:N]
